# baseline (speedup 1.0000x reference)
import functools

import jax
import jax.numpy as jnp
from jax import lax
from jax.experimental import pallas as pl
from jax.experimental.pallas import tpu as pltpu
from jax.experimental.pallas import tpu_sc as plsc

_VOCAB = 1000000
_DIM = 64
_B = 16384
_L = 20
_EPS = 1e-5

_NROWS = _B * _L
_NWORKERS = 32
_ROWS_PER_W = _NROWS // _NWORKERS
_IDXW = 128
_CHUNK = 1024
_BATCHES = _CHUNK // _IDXW
_NCHUNKS = _ROWS_PER_W // _CHUNK
_IDX_ROWS = _ROWS_PER_W // _IDXW


def _rsqrt16(v):
    vi = plsc.bitcast(v, jnp.int32)
    yi = jnp.int32(0x5F3759DF) - lax.shift_right_arithmetic(vi, jnp.int32(1))
    y = plsc.bitcast(yi, jnp.float32)
    half_v = v * 0.5
    for _ in range(3):
        y = y * (1.5 - half_v * y * y)
    return y


def _sc_body(x_hbm, table_hbm, gamma_hbm, beta_hbm, out_hbm,
             idx_v, rows_v, gamma_v, beta_v, sem):
    cid = lax.axis_index("c")
    sid = lax.axis_index("s")
    wid = sid * 2 + cid

    pltpu.sync_copy(x_hbm.at[wid], idx_v)
    pltpu.sync_copy(gamma_hbm, gamma_v)
    pltpu.sync_copy(beta_hbm, beta_v)

    lane = lax.iota(jnp.int32, 16)
    zeros = jnp.zeros((16,), jnp.float32)
    base_row = wid * _ROWS_PER_W

    def chunk_body(ci, carry):
        descs = []
        for b in range(_BATCHES):
            d = pltpu.make_async_copy(
                table_hbm.at[idx_v.at[ci * _BATCHES + b]],
                rows_v.at[pl.ds(b * _IDXW, _IDXW)],
                sem,
            )
            d.start()
            descs.append(d)
        for d in descs:
            d.wait()

        def group_body(g, c2):
            ridx = g * 16 + lane

            def col_sum(j, acc):
                sm, sq = acc
                cidx = jnp.full((16,), j, jnp.int32)
                col = plsc.load_gather(rows_v, [ridx, cidx])
                return sm + col, sq + col * col

            sm, sq = lax.fori_loop(0, _DIM, col_sum, (zeros, zeros))
            mean = sm * (1.0 / _DIM)
            var = sq * (1.0 / _DIM) - mean * mean
            inv = _rsqrt16(var + _EPS)

            def col_norm(j, c3):
                cidx = jnp.full((16,), j, jnp.int32)
                col = plsc.load_gather(rows_v, [ridx, cidx])
                gj = plsc.load_gather(gamma_v, [cidx])
                bj = plsc.load_gather(beta_v, [cidx])
                res = (col - mean) * (inv * gj) + bj
                plsc.store_scatter(rows_v, [ridx, cidx], res)
                return c3

            return lax.fori_loop(0, _DIM, col_norm, c2)

        lax.fori_loop(0, _CHUNK // 16, group_body, 0)

        pltpu.sync_copy(
            rows_v, out_hbm.at[pl.ds(base_row + ci * _CHUNK, _CHUNK)])
        return carry

    lax.fori_loop(0, _NCHUNKS, chunk_body, 0)


@jax.jit
def kernel(x, table, gamma, beta):
    idx = x.astype(jnp.int32).reshape(_NWORKERS, _IDX_ROWS, _IDXW)
    mesh = plsc.VectorSubcoreMesh(core_axis_name="c", subcore_axis_name="s")
    out = pl.kernel(
        _sc_body,
        out_type=jax.ShapeDtypeStruct((_NROWS, _DIM), jnp.float32),
        mesh=mesh,
        compiler_params=pltpu.CompilerParams(
            needs_layout_passes=False, use_tc_tiling_on_sc=False),
        scratch_types=[
            pltpu.VMEM((_IDX_ROWS, _IDXW), jnp.int32),
            pltpu.VMEM((_CHUNK, _DIM), jnp.float32),
            pltpu.VMEM((_DIM,), jnp.float32),
            pltpu.VMEM((_DIM,), jnp.float32),
            pltpu.SemaphoreType.DMA,
        ],
    )(idx, table, gamma, beta)
    return out.reshape(_B, _L, _DIM)


# speedup vs baseline: 1.5692x; 1.5692x over previous
import functools

import jax
import jax.numpy as jnp
from jax import lax
from jax.experimental import pallas as pl
from jax.experimental.pallas import tpu as pltpu
from jax.experimental.pallas import tpu_sc as plsc

_VOCAB = 1000000
_DIM = 64
_B = 16384
_L = 20
_EPS = 1e-5

_NROWS = _B * _L
_NWORKERS = 32
_ROWS_PER_W = _NROWS // _NWORKERS
_IDXW = 128
_CHUNK = 1024
_BATCHES = _CHUNK // _IDXW
_NCHUNKS = _ROWS_PER_W // _CHUNK
_IDX_ROWS = _ROWS_PER_W // _IDXW


def _rsqrt16(v):
    vi = plsc.bitcast(v, jnp.int32)
    yi = jnp.int32(0x5F3759DF) - lax.shift_right_arithmetic(vi, jnp.int32(1))
    y = plsc.bitcast(yi, jnp.float32)
    half_v = v * 0.5
    for _ in range(3):
        y = y * (1.5 - half_v * y * y)
    return y


def _sc_body(x_hbm, table_hbm, gamma_hbm, beta_hbm, out_hbm,
             idx_v, rows_v, gamma_v, beta_v, sem):
    cid = lax.axis_index("c")
    sid = lax.axis_index("s")
    wid = sid * 2 + cid

    pltpu.sync_copy(x_hbm.at[wid], idx_v)
    pltpu.sync_copy(gamma_hbm, gamma_v)
    pltpu.sync_copy(beta_hbm, beta_v)

    lane = lax.iota(jnp.int32, 16)
    zeros = jnp.zeros((16,), jnp.float32)
    base_row = wid * _ROWS_PER_W

    gs = [gamma_v[pl.ds(k * 16, 16)] for k in range(4)]
    bs = [beta_v[pl.ds(k * 16, 16)] for k in range(4)]
    cidx = [jnp.full((16,), j, jnp.int32) for j in range(_DIM)]
    rfull = [jnp.full((16,), r, jnp.int32) for r in range(16)]

    def chunk_body(ci, carry):
        descs = []
        for b in range(_BATCHES):
            d = pltpu.make_async_copy(
                table_hbm.at[idx_v.at[ci * _BATCHES + b]],
                rows_v.at[pl.ds(b * _IDXW, _IDXW)],
                sem,
            )
            d.start()
            descs.append(d)
        for d in descs:
            d.wait()

        def group_body(g, c2):
            base = g * 16
            ridx = base + lane
            sm = [zeros] * 4
            sq = [zeros] * 4
            for j in range(_DIM):
                col = plsc.load_gather(rows_v, [ridx, cidx[j]])
                a = j % 4
                sm[a] = sm[a] + col
                sq[a] = sq[a] + col * col
            smt = (sm[0] + sm[1]) + (sm[2] + sm[3])
            sqt = (sq[0] + sq[1]) + (sq[2] + sq[3])
            mean = smt * (1.0 / _DIM)
            var = sqt * (1.0 / _DIM) - mean * mean
            inv = _rsqrt16(var + _EPS)
            shift = -mean * inv
            for r in range(16):
                s_r = inv.at[rfull[r]].get(mode="promise_in_bounds")
                t_r = shift.at[rfull[r]].get(mode="promise_in_bounds")
                row = base + r
                for k in range(4):
                    v = rows_v[row, pl.ds(k * 16, 16)]
                    rows_v[row, pl.ds(k * 16, 16)] = (
                        (v * s_r + t_r) * gs[k] + bs[k])
            return c2

        lax.fori_loop(0, _CHUNK // 16, group_body, 0)

        pltpu.sync_copy(
            rows_v, out_hbm.at[pl.ds(base_row + ci * _CHUNK, _CHUNK)])
        return carry

    lax.fori_loop(0, _NCHUNKS, chunk_body, 0)


@jax.jit
def kernel(x, table, gamma, beta):
    idx = x.astype(jnp.int32).reshape(_NWORKERS, _IDX_ROWS, _IDXW)
    mesh = plsc.VectorSubcoreMesh(core_axis_name="c", subcore_axis_name="s")
    out = pl.kernel(
        _sc_body,
        out_type=jax.ShapeDtypeStruct((_NROWS, _DIM), jnp.float32),
        mesh=mesh,
        compiler_params=pltpu.CompilerParams(
            needs_layout_passes=False, use_tc_tiling_on_sc=False),
        scratch_types=[
            pltpu.VMEM((_IDX_ROWS, _IDXW), jnp.int32),
            pltpu.VMEM((_CHUNK, _DIM), jnp.float32),
            pltpu.VMEM((_DIM,), jnp.float32),
            pltpu.VMEM((_DIM,), jnp.float32),
            pltpu.SemaphoreType.DMA,
        ],
    )(idx, table, gamma, beta)
    return out.reshape(_B, _L, _DIM)


# speedup vs baseline: 1.8619x; 1.1865x over previous
import jax
import jax.numpy as jnp
from jax import lax
from jax.experimental import pallas as pl
from jax.experimental.pallas import tpu as pltpu
from jax.experimental.pallas import tpu_sc as plsc

_DIM = 64
_B = 16384
_L = 20
_EPS = 1e-5

_NROWS = _B * _L
_NWORKERS = 32
_ROWS_PER_W = _NROWS // _NWORKERS
_IDXW = 128
_CHUNK = 512
_BATCHES = _CHUNK // _IDXW
_NCHUNKS = _ROWS_PER_W // _CHUNK
_IDX_ROWS = _ROWS_PER_W // _IDXW
_UNROLL = 4


def _rsqrt16(v):
    vi = plsc.bitcast(v, jnp.int32)
    yi = jnp.int32(0x5F3759DF) - lax.shift_right_arithmetic(vi, jnp.int32(1))
    y = plsc.bitcast(yi, jnp.float32)
    half_v = v * 0.5
    for _ in range(2):
        y = y * (1.5 - half_v * y * y)
    return y


def _start_gathers(table_hbm, idx_v, rows3, sem, ci):
    slot = lax.rem(ci, 2)
    for b in range(_BATCHES):
        pltpu.make_async_copy(
            table_hbm.at[idx_v.at[ci * _BATCHES + b]],
            rows3.at[slot, pl.ds(b * _IDXW, _IDXW)],
            sem,
        ).start()


def _wait_gathers(table_hbm, idx_v, rows3, sem, ci):
    slot = lax.rem(ci, 2)
    for b in range(_BATCHES):
        pltpu.make_async_copy(
            table_hbm.at[idx_v.at[ci * _BATCHES + b]],
            rows3.at[slot, pl.ds(b * _IDXW, _IDXW)],
            sem,
        ).wait()


def _out_desc(out_hbm, rows3, sem, ci, base_row):
    slot = lax.rem(ci, 2)
    return pltpu.make_async_copy(
        rows3.at[slot],
        out_hbm.at[pl.ds(base_row + ci * _CHUNK, _CHUNK)],
        sem,
    )


def _sc_body(x_hbm, table_hbm, gamma_hbm, beta_hbm, out_hbm,
             idx_v, rows3, gamma_v, beta_v, sem_g, sem_o):
    cid = lax.axis_index("c")
    sid = lax.axis_index("s")
    wid = sid * 2 + cid

    pltpu.sync_copy(x_hbm.at[wid], idx_v)
    pltpu.sync_copy(gamma_hbm, gamma_v)
    pltpu.sync_copy(beta_hbm, beta_v)

    base_row = wid * _ROWS_PER_W
    gs = [gamma_v[pl.ds(k * 16, 16)] for k in range(4)]
    bs = [beta_v[pl.ds(k * 16, 16)] for k in range(4)]
    last = jnp.full((16,), 15, jnp.int32)

    _start_gathers(table_hbm, idx_v, rows3, sem_g, 0)

    def chunk_body(ci, carry):
        @pl.when(ci >= 1)
        def _():
            _out_desc(out_hbm, rows3, sem_o, ci - 1, base_row).wait()

        @pl.when(ci + 1 < _NCHUNKS)
        def _():
            _start_gathers(table_hbm, idx_v, rows3, sem_g, ci + 1)

        _wait_gathers(table_hbm, idx_v, rows3, sem_g, ci)
        slot = lax.rem(ci, 2)
        buf = rows3.at[slot]

        def rows_body(it, c2):
            for u in range(_UNROLL):
                row = it * _UNROLL + u
                v = [buf[row, pl.ds(k * 16, 16)] for k in range(4)]
                s = (v[0] + v[1]) + (v[2] + v[3])
                q = ((v[0] * v[0] + v[1] * v[1])
                     + (v[2] * v[2] + v[3] * v[3]))
                tot = plsc.cumsum(s).at[last].get(mode="promise_in_bounds")
                totq = plsc.cumsum(q).at[last].get(mode="promise_in_bounds")
                mean = tot * (1.0 / _DIM)
                var = totq * (1.0 / _DIM) - mean * mean
                inv = _rsqrt16(var + _EPS)
                shift = -mean * inv
                for k in range(4):
                    buf[row, pl.ds(k * 16, 16)] = (
                        (v[k] * inv + shift) * gs[k] + bs[k])
            return c2

        lax.fori_loop(0, _CHUNK // _UNROLL, rows_body, 0)

        _out_desc(out_hbm, rows3, sem_o, ci, base_row).start()
        return carry

    lax.fori_loop(0, _NCHUNKS, chunk_body, 0)
    _out_desc(out_hbm, rows3, sem_o, _NCHUNKS - 1, base_row).wait()


@jax.jit
def kernel(x, table, gamma, beta):
    idx = x.astype(jnp.int32).reshape(_NWORKERS, _IDX_ROWS, _IDXW)
    mesh = plsc.VectorSubcoreMesh(core_axis_name="c", subcore_axis_name="s")
    out = pl.kernel(
        _sc_body,
        out_type=jax.ShapeDtypeStruct((_NROWS, _DIM), jnp.float32),
        mesh=mesh,
        compiler_params=pltpu.CompilerParams(
            needs_layout_passes=False, use_tc_tiling_on_sc=False),
        scratch_types=[
            pltpu.VMEM((_IDX_ROWS, _IDXW), jnp.int32),
            pltpu.VMEM((2, _CHUNK, _DIM), jnp.float32),
            pltpu.VMEM((_DIM,), jnp.float32),
            pltpu.VMEM((_DIM,), jnp.float32),
            pltpu.SemaphoreType.DMA,
            pltpu.SemaphoreType.DMA,
        ],
    )(idx, table, gamma, beta)
    return out.reshape(_B, _L, _DIM)
